# baseline (speedup 1.0000x reference)
import functools

import jax
import jax.numpy as jnp
from jax import lax
from jax.experimental import pallas as pl
from jax.experimental.pallas import tpu as pltpu
from jax.experimental.pallas import tpu_sc as plsc

N = 10000
E = 320000
EP = 100000
DIN, DH, DOUT = 128, 64, 32
NP = 10240
NC, NS = 2, 16
NW = NC * NS
RT = NP // NS
ECJ = 125
EJ = 8
EC = ECJ * EJ
EW = E // NW
NCH = EW // EC
ERWS = E // ECJ
PRWS = EP // ECJ
PCH = PRWS // EJ
RB = 2048
NB = NP // RB

_MESH = plsc.VectorSubcoreMesh(
    core_axis_name="c", subcore_axis_name="s", num_cores=NC, num_subcores=NS)
_SC_PARAMS = pltpu.CompilerParams(use_tc_tiling_on_sc=False)
_SC_PARAMS_NL = pltpu.CompilerParams(use_tc_tiling_on_sc=False,
                                     needs_layout_passes=False)

_f32 = jnp.float32
_i32 = jnp.int32


@functools.partial(
    pl.kernel,
    out_type=jax.ShapeDtypeStruct((NC, 6, NP), _f32),
    mesh=_MESH,
    compiler_params=_SC_PARAMS,
    scratch_types=[
        pltpu.VMEM((1024,), _f32),
        pltpu.VMEM((RT,), _f32),
    ] + [pltpu.VMEM((EC,), _i32) for _ in range(12)]
      + [pltpu.VMEM_SHARED((NP,), _f32) for _ in range(6)]
      + [pltpu.SemaphoreType.DMA, pltpu.SemaphoreType.DMA],
)
def _deg_kernel(e0s, e0d, e1s, e1d, e2s, e2d, zvec, out, ones, stage, *rest):
    idxab = rest[:12]
    shs = rest[12:18]
    sa, sb = rest[18:]
    cid = lax.axis_index("c")
    sid = lax.axis_index("s")
    wid = cid * NS + sid
    for i in range(64):
        ones[pl.ds(16 * i, 16)] = jnp.full((16,), 1.0, _f32)
    ro0 = pl.multiple_of(sid * RT, 8)
    pltpu.sync_copy(zvec, stage)
    for sh in shs:
        pltpu.sync_copy(stage, sh.at[pl.ds(ro0, RT)])
    plsc.subcore_barrier()
    eisrc = (e0s, e0d, e1s, e1d, e2s, e2d)

    def halfchunk(k, bufs, ssem, drain_first):
        eo = pl.multiple_of(wid * EW + k * EC, 8)
        if drain_first:
            for p in range(6):
                pltpu.make_async_copy(ones.at[pl.ds(0, EC)],
                                      shs[p].at[bufs[p]], ssem).wait()
        for p in range(6):
            pltpu.sync_copy(eisrc[p].at[pl.ds(eo, EC)], bufs[p])
        for p in range(6):
            pltpu.async_copy(ones.at[pl.ds(0, EC)], shs[p].at[bufs[p]], ssem,
                             add=True)

    def chunk2(k2, carry):
        halfchunk(2 * k2, idxab[:6], sa, True)
        halfchunk(2 * k2 + 1, idxab[6:], sb, True)
        return carry

    halfchunk(0, idxab[:6], sa, False)
    halfchunk(1, idxab[6:], sb, False)

    def chunk2w(k2, carry):
        return chunk2(k2 + 1, carry)

    lax.fori_loop(0, NCH // 2 - 1, chunk2w, 0)
    for bufs, ssem in ((idxab[:6], sa), (idxab[6:], sb)):
        for p in range(6):
            pltpu.make_async_copy(ones.at[pl.ds(0, EC)],
                                  shs[p].at[bufs[p]], ssem).wait()
    plsc.subcore_barrier()
    for jj, sh in enumerate(shs):
        pltpu.sync_copy(sh.at[pl.ds(ro0, RT)], stage)
        pltpu.sync_copy(stage, out.at[cid, jj, pl.ds(ro0, RT)])


def _make_scatter(D, ei_of):
    ntab = len(ei_of)

    @functools.partial(
        pl.kernel,
        out_type=jax.ShapeDtypeStruct((NC, ntab, NP, D), _f32),
        mesh=_MESH,
        compiler_params=_SC_PARAMS,
        scratch_types=[
            pltpu.VMEM((EC,), _i32),
            pltpu.VMEM((EC,), _i32),
            pltpu.VMEM((EC, D), _f32),
            pltpu.VMEM((EC,), _i32),
            pltpu.VMEM((EC,), _i32),
            pltpu.VMEM((EC, D), _f32),
            pltpu.VMEM((RT, D), _f32),
            pltpu.VMEM_SHARED((NP, D), _f32),
            pltpu.SemaphoreType.DMA,
            pltpu.SemaphoreType.DMA,
            pltpu.SemaphoreType.DMA,
            pltpu.SemaphoreType.DMA,
        ],
    )
    def k(tabs3, e0s, e0d, e1s, e1d, e2s, e2d, zrows, acc_out, *scr):
        tabs = [tabs3.at[t] for t in range(ntab)]
        eis = ((e0s, e0d), (e1s, e1d), (e2s, e2d))
        (isa, ida, rwa, isb, idb, rwb, stage, ash,
         gsa, gsb, ssa, ssb) = scr
        cid = lax.axis_index("c")
        sid = lax.axis_index("s")
        wid = cid * NS + sid
        ro0 = pl.multiple_of(sid * RT, 8)

        def load_idx(er, isx, idx_, k_):
            eo = pl.multiple_of(wid * EW + k_ * EC, 8)
            pltpu.sync_copy(er[0].at[pl.ds(eo, EC)], isx)
            pltpu.sync_copy(er[1].at[pl.ds(eo, EC)], idx_)

        def fire_g(tr, isx, rw, gs):
            pltpu.async_copy(tr.at[isx], rw, gs)

        def drain_g(tr, isx, rw, gs):
            pltpu.make_async_copy(tr.at[isx], rw, gs).wait()

        def fire_s(idx_, rw, ss):
            pltpu.async_copy(rw, ash.at[idx_], ss, add=True)

        def drain_s(idx_, rw, ss):
            pltpu.make_async_copy(rw, ash.at[idx_], ss).wait()

        for t in range(ntab):
            tr = tabs[t]
            er = eis[ei_of[t]]
            load_idx(er, isa, ida, 0)
            fire_g(tr, isa, rwa, gsa)
            pltpu.sync_copy(zrows, stage)
            pltpu.sync_copy(stage, ash.at[pl.ds(ro0, RT), :])
            plsc.subcore_barrier()

            def body(k2, carry, tr=tr, er=er):
                k_ = 2 * k2
                load_idx(er, isb, idb, k_ + 1)
                drain_g(tr, isa, rwa, gsa)
                fire_s(ida, rwa, ssa)
                fire_g(tr, isb, rwb, gsb)
                drain_s(ida, rwa, ssa)

                @pl.when(k_ + 2 < NCH)
                def _():
                    load_idx(er, isa, ida, k_ + 2)
                    fire_g(tr, isa, rwa, gsa)

                drain_g(tr, isb, rwb, gsb)
                fire_s(idb, rwb, ssb)
                drain_s(idb, rwb, ssb)
                return carry

            lax.fori_loop(0, NCH // 2, body, 0)
            plsc.subcore_barrier()
            pltpu.sync_copy(ash.at[pl.ds(ro0, RT), :], stage)
            pltpu.sync_copy(stage, acc_out.at[cid, t, pl.ds(ro0, RT), :])

    return k


_scatter_l1 = _make_scatter(DOUT, (0, 0, 1, 1, 2, 2))
_scatter_l2 = _make_scatter(DOUT, (0, 1, 2))


@functools.partial(
    pl.kernel,
    out_type=(jax.ShapeDtypeStruct((EP, DOUT), _f32),
              jax.ShapeDtypeStruct((EP, DOUT), _f32)),
    mesh=_MESH,
    compiler_params=_SC_PARAMS,
    scratch_types=[
        pltpu.VMEM((EC,), _i32),
        pltpu.VMEM((EC,), _i32),
        pltpu.VMEM((EC, DOUT), _f32),
        pltpu.VMEM((EC, DOUT), _f32),
        pltpu.SemaphoreType.DMA,
    ],
)
def _pred_gather(h2, pu, pv, u_out, v_out, idxu, idxv, ur, vr, sem):
    cid = lax.axis_index("c")
    sid = lax.axis_index("s")
    wid = cid * NS + sid

    def do_chunk(kk):
        eo = pl.multiple_of((wid + NW * kk) * EC, 8)
        pltpu.sync_copy(pu.at[pl.ds(eo, EC)], idxu)
        pltpu.sync_copy(pv.at[pl.ds(eo, EC)], idxv)
        cps = [pltpu.async_copy(h2.at[idxu], ur, sem),
               pltpu.async_copy(h2.at[idxv], vr, sem)]
        for c in cps:
            c.wait()
        pltpu.sync_copy(ur, u_out.at[pl.ds(eo, EC), :])
        pltpu.sync_copy(vr, v_out.at[pl.ds(eo, EC), :])

    for kk in range(4):
        if (kk + 1) * NW <= PCH:
            do_chunk(kk)
        else:
            @pl.when(wid + NW * kk < PCH)
            def _():
                do_chunk(kk)


EPQ = EP // 4
RBD = 5000


def _dotp_body(u_ref, v_ref, w_ref, o_ref):
    o_ref[...] = jnp.dot(u_ref[...] * v_ref[...], w_ref[...],
                         preferred_element_type=_f32,
                         precision=lax.Precision.HIGHEST)


_dotp = pl.pallas_call(
    _dotp_body,
    grid=(EPQ // RBD,),
    in_specs=[
        pl.BlockSpec((RBD, 128), lambda i: (i, 0)),
        pl.BlockSpec((RBD, 128), lambda i: (i, 0)),
        pl.BlockSpec((128, 4), lambda i: (0, 0)),
    ],
    out_specs=pl.BlockSpec((RBD, 4), lambda i: (i, 0)),
    out_shape=jax.ShapeDtypeStruct((EPQ, 4), _f32),
)


def _mm1_body(x_ref, dc_ref, w_ref, o_ref):
    for r in range(3):
        ns = lax.rsqrt(jnp.maximum(dc_ref[:, 2 * r:2 * r + 1], 1.0))
        xs = x_ref[...] * ns
        for h in range(2):
            o_ref[2 * r + h] = jnp.dot(
                xs, w_ref[r, :, h * DOUT:(h + 1) * DOUT],
                preferred_element_type=_f32)


_mm1 = pl.pallas_call(
    _mm1_body,
    grid=(NB,),
    in_specs=[
        pl.BlockSpec((RB, DIN), lambda i: (i, 0)),
        pl.BlockSpec((RB, 6), lambda i: (i, 0)),
        pl.BlockSpec((3, DIN, DH), lambda i: (0, 0, 0)),
    ],
    out_specs=pl.BlockSpec((6, RB, DOUT), lambda i: (0, i, 0)),
    out_shape=jax.ShapeDtypeStruct((6, NP, DOUT), _f32),
)


NPQ = NP // 4
RBQ = 256


def _cmb1_body(acc_ref, dgm_ref, b1_ref, w2_ref, o_ref):
    hp = []
    for half in range(2):
        h = jnp.zeros((RBQ, 128), _f32)
        for r in range(3):
            nd = lax.rsqrt(jnp.maximum(dgm_ref[2 * r + 1], 1.0))
            h = h + (acc_ref[0, 2 * r + half] + acc_ref[1, 2 * r + half]) \
                * nd + b1_ref[r, half][None, :]
        hp.append(jnp.maximum(h, 0.0))
    for ro in range(3):
        ns = lax.rsqrt(jnp.maximum(dgm_ref[2 * ro], 1.0))
        o_ref[ro] = (
            jnp.dot(hp[0] * ns, w2_ref[ro, 0], preferred_element_type=_f32)
            + jnp.dot(hp[1] * ns, w2_ref[ro, 1], preferred_element_type=_f32))


_cmb1 = pl.pallas_call(
    _cmb1_body,
    grid=(NPQ // RBQ,),
    in_specs=[
        pl.BlockSpec((NC, 6, RBQ, 128), lambda i: (0, 0, i, 0)),
        pl.BlockSpec((6, RBQ, 128), lambda i: (0, i, 0)),
        pl.BlockSpec((3, 2, 128), lambda i: (0, 0, 0)),
        pl.BlockSpec((3, 2, 128, 128), lambda i: (0, 0, 0, 0)),
    ],
    out_specs=pl.BlockSpec((3, RBQ, 128), lambda i: (0, i, 0)),
    out_shape=jax.ShapeDtypeStruct((3, NPQ, 128), _f32),
)


def _cmb2_body(acc_ref, dgm_ref, b2_ref, o_ref):
    h2 = jnp.zeros((RBQ, 128), _f32)
    for r in range(3):
        nd = lax.rsqrt(jnp.maximum(dgm_ref[2 * r + 1], 1.0))
        h2 = h2 + (acc_ref[0, r] + acc_ref[1, r]) * nd + b2_ref[r][None, :]
    o_ref[...] = h2


_cmb2 = pl.pallas_call(
    _cmb2_body,
    grid=(NPQ // RBQ,),
    in_specs=[
        pl.BlockSpec((NC, 3, RBQ, 128), lambda i: (0, 0, i, 0)),
        pl.BlockSpec((6, RBQ, 128), lambda i: (0, i, 0)),
        pl.BlockSpec((3, 128), lambda i: (0, 0)),
    ],
    out_specs=pl.BlockSpec((RBQ, 128), lambda i: (i, 0)),
    out_shape=jax.ShapeDtypeStruct((NPQ, 128), _f32),
)


def kernel(x, edge_index_rel0, edge_index_rel1, edge_index_rel2,
           pred_edge_index,
           W1_0, b1_0, W1_1, b1_1, W1_2, b1_2,
           W2_0, b2_0, W2_1, b2_1, W2_2, b2_2):
    e0s, e0d = edge_index_rel0[0], edge_index_rel0[1]
    e1s, e1d = edge_index_rel1[0], edge_index_rel1[1]
    e2s, e2d = edge_index_rel2[0], edge_index_rel2[1]
    eis = (e0s, e0d, e1s, e1d, e2s, e2d)
    zvec = jnp.zeros((RT,), _f32)
    degp = _deg_kernel(*eis, zvec)
    deg6 = degp[0] + degp[1]
    degcol = deg6.T
    degm = jnp.broadcast_to(deg6[:, :, None],
                            (6, NP, 32)).reshape(6, NPQ, 128)

    xp = jnp.pad(x, ((0, NP - N), (0, 0)))
    w1s = jnp.stack([W1_0, W1_1, W1_2])
    t1 = _mm1(xp, degcol, w1s)

    z32 = jnp.zeros((RT, DOUT), _f32)
    acc1 = _scatter_l1(t1, *eis, z32)

    eye4 = jnp.eye(4, dtype=_f32)
    b1p = jnp.stack([jnp.stack([jnp.tile(b[:DOUT], 4), jnp.tile(b[DOUT:], 4)])
                     for b in (b1_0, b1_1, b1_2)])
    w2bd = jnp.stack([jnp.stack([jnp.kron(eye4, w[:DOUT]),
                                 jnp.kron(eye4, w[DOUT:])])
                      for w in (W2_0, W2_1, W2_2)])
    acc1v = acc1.reshape(NC, 6, NPQ, 128)
    t2p = _cmb1(acc1v, degm, b1p, w2bd)
    t2 = t2p.reshape(3, NP, DOUT)

    acc2 = _scatter_l2(t2, *eis, z32)

    b2p = jnp.stack([jnp.tile(b, 4) for b in (b2_0, b2_1, b2_2)])
    acc2v = acc2.reshape(NC, 3, NPQ, 128)
    h2p = _cmb2(acc2v, degm, b2p)
    h2 = h2p.reshape(NP, DOUT)

    u_rows, v_rows = _pred_gather(h2, pred_edge_index[0],
                                  pred_edge_index[1])
    ones4 = jnp.kron(jnp.eye(4, dtype=_f32), jnp.ones((DOUT, 1), _f32))
    scp = _dotp(u_rows.reshape(EPQ, 128), v_rows.reshape(EPQ, 128), ones4)
    return scp.reshape(EP, 1)


# speedup vs baseline: 1.0118x; 1.0118x over previous
import functools

import jax
import jax.numpy as jnp
from jax import lax
from jax.experimental import pallas as pl
from jax.experimental.pallas import tpu as pltpu
from jax.experimental.pallas import tpu_sc as plsc

N = 10000
E = 320000
EP = 100000
DIN, DH, DOUT = 128, 64, 32
NP = 10240
NC, NS = 2, 16
NW = NC * NS
RT = NP // NS
ECJ = 125
EJ = 8
EC = ECJ * EJ
EW = E // NW
NCH = EW // EC
ERWS = E // ECJ
PRWS = EP // ECJ
PCH = PRWS // EJ
RB = 2048
NB = NP // RB

_MESH = plsc.VectorSubcoreMesh(
    core_axis_name="c", subcore_axis_name="s", num_cores=NC, num_subcores=NS)
_SC_PARAMS = pltpu.CompilerParams(use_tc_tiling_on_sc=False)
_SC_PARAMS_NL = pltpu.CompilerParams(use_tc_tiling_on_sc=False,
                                     needs_layout_passes=False)

_f32 = jnp.float32
_i32 = jnp.int32


@functools.partial(
    pl.kernel,
    out_type=jax.ShapeDtypeStruct((NC, 6, NP), _f32),
    mesh=_MESH,
    compiler_params=_SC_PARAMS,
    scratch_types=[
        pltpu.VMEM((1024,), _f32),
        pltpu.VMEM((RT,), _f32),
    ] + [pltpu.VMEM((EC,), _i32) for _ in range(12)]
      + [pltpu.VMEM_SHARED((NP,), _f32) for _ in range(6)]
      + [pltpu.SemaphoreType.DMA, pltpu.SemaphoreType.DMA],
)
def _deg_kernel(e0s, e0d, e1s, e1d, e2s, e2d, zvec, out, ones, stage, *rest):
    idxab = rest[:12]
    shs = rest[12:18]
    sa, sb = rest[18:]
    cid = lax.axis_index("c")
    sid = lax.axis_index("s")
    wid = cid * NS + sid
    for i in range(64):
        ones[pl.ds(16 * i, 16)] = jnp.full((16,), 1.0, _f32)
    ro0 = pl.multiple_of(sid * RT, 8)
    pltpu.sync_copy(zvec, stage)
    for sh in shs:
        pltpu.sync_copy(stage, sh.at[pl.ds(ro0, RT)])
    plsc.subcore_barrier()
    eisrc = (e0s, e0d, e1s, e1d, e2s, e2d)

    def halfchunk(k, bufs, ssem, drain_first):
        eo = pl.multiple_of(wid * EW + k * EC, 8)
        if drain_first:
            for p in range(6):
                pltpu.make_async_copy(ones.at[pl.ds(0, EC)],
                                      shs[p].at[bufs[p]], ssem).wait()
        for p in range(6):
            pltpu.sync_copy(eisrc[p].at[pl.ds(eo, EC)], bufs[p])
        for p in range(6):
            pltpu.async_copy(ones.at[pl.ds(0, EC)], shs[p].at[bufs[p]], ssem,
                             add=True)

    def chunk2(k2, carry):
        halfchunk(2 * k2, idxab[:6], sa, True)
        halfchunk(2 * k2 + 1, idxab[6:], sb, True)
        return carry

    halfchunk(0, idxab[:6], sa, False)
    halfchunk(1, idxab[6:], sb, False)

    def chunk2w(k2, carry):
        return chunk2(k2 + 1, carry)

    lax.fori_loop(0, NCH // 2 - 1, chunk2w, 0)
    for bufs, ssem in ((idxab[:6], sa), (idxab[6:], sb)):
        for p in range(6):
            pltpu.make_async_copy(ones.at[pl.ds(0, EC)],
                                  shs[p].at[bufs[p]], ssem).wait()
    plsc.subcore_barrier()
    for jj, sh in enumerate(shs):
        pltpu.sync_copy(sh.at[pl.ds(ro0, RT)], stage)
        pltpu.sync_copy(stage, out.at[cid, jj, pl.ds(ro0, RT)])


def _make_scatter(D, ei_of):
    ntab = len(ei_of)

    @functools.partial(
        pl.kernel,
        out_type=jax.ShapeDtypeStruct((NC, ntab, NP, D), _f32),
        mesh=_MESH,
        compiler_params=_SC_PARAMS,
        scratch_types=[
            pltpu.VMEM((EC,), _i32),
            pltpu.VMEM((EC,), _i32),
            pltpu.VMEM((EC, D), _f32),
            pltpu.VMEM((EC,), _i32),
            pltpu.VMEM((EC,), _i32),
            pltpu.VMEM((EC, D), _f32),
            pltpu.VMEM((RT, D), _f32),
            pltpu.VMEM_SHARED((NP, D), _f32),
            pltpu.SemaphoreType.DMA,
            pltpu.SemaphoreType.DMA,
            pltpu.SemaphoreType.DMA,
            pltpu.SemaphoreType.DMA,
        ],
    )
    def k(tabs3, e0s, e0d, e1s, e1d, e2s, e2d, zrows, acc_out, *scr):
        tabs = [tabs3.at[t] for t in range(ntab)]
        eis = ((e0s, e0d), (e1s, e1d), (e2s, e2d))
        (isa, ida, rwa, isb, idb, rwb, stage, ash,
         gsa, gsb, ssa, ssb) = scr
        cid = lax.axis_index("c")
        sid = lax.axis_index("s")
        wid = cid * NS + sid
        ro0 = pl.multiple_of(sid * RT, 8)

        def load_idx(er, isx, idx_, k_):
            eo = pl.multiple_of(wid * EW + k_ * EC, 8)
            pltpu.sync_copy(er[0].at[pl.ds(eo, EC)], isx)
            pltpu.sync_copy(er[1].at[pl.ds(eo, EC)], idx_)

        def fire_g(tr, isx, rw, gs):
            pltpu.async_copy(tr.at[isx], rw, gs)

        def drain_g(tr, isx, rw, gs):
            pltpu.make_async_copy(tr.at[isx], rw, gs).wait()

        def fire_s(idx_, rw, ss):
            pltpu.async_copy(rw, ash.at[idx_], ss, add=True)

        def drain_s(idx_, rw, ss):
            pltpu.make_async_copy(rw, ash.at[idx_], ss).wait()

        for t in range(ntab):
            tr = tabs[t]
            er = eis[ei_of[t]]
            load_idx(er, isa, ida, 0)
            fire_g(tr, isa, rwa, gsa)
            pltpu.sync_copy(zrows, stage)
            pltpu.sync_copy(stage, ash.at[pl.ds(ro0, RT), :])
            plsc.subcore_barrier()

            def body(k2, carry, tr=tr, er=er):
                k_ = 2 * k2
                load_idx(er, isb, idb, k_ + 1)
                drain_g(tr, isa, rwa, gsa)
                fire_s(ida, rwa, ssa)
                fire_g(tr, isb, rwb, gsb)
                drain_s(ida, rwa, ssa)

                @pl.when(k_ + 2 < NCH)
                def _():
                    load_idx(er, isa, ida, k_ + 2)
                    fire_g(tr, isa, rwa, gsa)

                drain_g(tr, isb, rwb, gsb)
                fire_s(idb, rwb, ssb)
                drain_s(idb, rwb, ssb)
                return carry

            lax.fori_loop(0, NCH // 2, body, 0)
            plsc.subcore_barrier()
            pltpu.sync_copy(ash.at[pl.ds(ro0, RT), :], stage)
            pltpu.sync_copy(stage, acc_out.at[cid, t, pl.ds(ro0, RT), :])

    return k


_scatter_l1 = _make_scatter(DOUT, (0, 0, 1, 1, 2, 2))
_scatter_l2 = _make_scatter(DOUT, (0, 1, 2))


PH0 = 512
PH1 = EC - PH0


@functools.partial(
    pl.kernel,
    out_type=jax.ShapeDtypeStruct((EP,), _f32),
    mesh=_MESH,
    compiler_params=_SC_PARAMS_NL,
    scratch_types=[
        pltpu.VMEM((EC,), _i32),
        pltpu.VMEM((EC,), _i32),
        pltpu.VMEM((EC, DOUT), _f32),
        pltpu.VMEM((EC, DOUT), _f32),
        pltpu.VMEM((1024,), _f32),
        pltpu.SemaphoreType.DMA,
        pltpu.SemaphoreType.DMA,
    ],
)
def _pred_kernel(h2, pu, pv, out, idxu, idxv, ur, vr, sbuf, sm0, sm1):
    cid = lax.axis_index("c")
    sid = lax.axis_index("s")
    wid = cid * NS + sid
    lane15 = lax.iota(_i32, 16) == 15
    halves = ((0, PH0, sm0), (PH0, PH1, sm1))

    def gpair(st, ln, sm):
        pltpu.async_copy(h2.at[idxu.at[pl.ds(st, ln)]],
                         ur.at[pl.ds(st, ln), :], sm)
        pltpu.async_copy(h2.at[idxv.at[pl.ds(st, ln)]],
                         vr.at[pl.ds(st, ln), :], sm)

    def dpair(st, ln, sm):
        pltpu.make_async_copy(h2.at[idxu.at[pl.ds(st, ln)]],
                              ur.at[pl.ds(st, ln), :], sm).wait()
        pltpu.make_async_copy(h2.at[idxv.at[pl.ds(st, ln)]],
                              vr.at[pl.ds(st, ln), :], sm).wait()

    def do_chunk(kk):
        eo = pl.multiple_of((wid + NW * kk) * EC, 8)
        pltpu.sync_copy(pu.at[pl.ds(eo, EC)], idxu)
        pltpu.sync_copy(pv.at[pl.ds(eo, EC)], idxv)
        for st, ln, sm in halves:
            gpair(st, ln, sm)

        def edge4(i, carry):
            for q in range(4):
                e = 4 * i + q
                p = (ur[e, pl.ds(0, 16)] * vr[e, pl.ds(0, 16)]
                     + ur[e, pl.ds(16, 16)] * vr[e, pl.ds(16, 16)])
                cs = plsc.cumsum(p)
                plsc.store_scatter(sbuf, [jnp.full((16,), e, _i32)], cs,
                                   mask=lane15)
            return carry

        dpair(0, PH0, sm0)
        lax.fori_loop(0, PH0 // 4, edge4, 0)
        dpair(PH0, PH1, sm1)
        lax.fori_loop(PH0 // 4, EC // 4, edge4, 0)
        pltpu.sync_copy(sbuf.at[pl.ds(0, EC)], out.at[pl.ds(eo, EC)])

    for kk in range(4):
        if (kk + 1) * NW <= PCH:
            do_chunk(kk)
        else:
            @pl.when(wid + NW * kk < PCH)
            def _():
                do_chunk(kk)


def _mm1_body(x_ref, dc_ref, w_ref, o_ref):
    for r in range(3):
        ns = lax.rsqrt(jnp.maximum(dc_ref[:, 2 * r:2 * r + 1], 1.0))
        xs = x_ref[...] * ns
        for h in range(2):
            o_ref[2 * r + h] = jnp.dot(
                xs, w_ref[r, :, h * DOUT:(h + 1) * DOUT],
                preferred_element_type=_f32)


_mm1 = pl.pallas_call(
    _mm1_body,
    grid=(NB,),
    in_specs=[
        pl.BlockSpec((RB, DIN), lambda i: (i, 0)),
        pl.BlockSpec((RB, 6), lambda i: (i, 0)),
        pl.BlockSpec((3, DIN, DH), lambda i: (0, 0, 0)),
    ],
    out_specs=pl.BlockSpec((6, RB, DOUT), lambda i: (0, i, 0)),
    out_shape=jax.ShapeDtypeStruct((6, NP, DOUT), _f32),
)


NPQ = NP // 4
RBQ = 256


def _cmb1_body(acc_ref, dgm_ref, b1_ref, w2_ref, o_ref):
    hp = []
    for half in range(2):
        h = jnp.zeros((RBQ, 128), _f32)
        for r in range(3):
            nd = lax.rsqrt(jnp.maximum(dgm_ref[2 * r + 1], 1.0))
            h = h + (acc_ref[0, 2 * r + half] + acc_ref[1, 2 * r + half]) \
                * nd + b1_ref[r, half][None, :]
        hp.append(jnp.maximum(h, 0.0))
    for ro in range(3):
        ns = lax.rsqrt(jnp.maximum(dgm_ref[2 * ro], 1.0))
        o_ref[ro] = (
            jnp.dot(hp[0] * ns, w2_ref[ro, 0], preferred_element_type=_f32)
            + jnp.dot(hp[1] * ns, w2_ref[ro, 1], preferred_element_type=_f32))


_cmb1 = pl.pallas_call(
    _cmb1_body,
    grid=(NPQ // RBQ,),
    in_specs=[
        pl.BlockSpec((NC, 6, RBQ, 128), lambda i: (0, 0, i, 0)),
        pl.BlockSpec((6, RBQ, 128), lambda i: (0, i, 0)),
        pl.BlockSpec((3, 2, 128), lambda i: (0, 0, 0)),
        pl.BlockSpec((3, 2, 128, 128), lambda i: (0, 0, 0, 0)),
    ],
    out_specs=pl.BlockSpec((3, RBQ, 128), lambda i: (0, i, 0)),
    out_shape=jax.ShapeDtypeStruct((3, NPQ, 128), _f32),
)


def _cmb2_body(acc_ref, dgm_ref, b2_ref, o_ref):
    h2 = jnp.zeros((RBQ, 128), _f32)
    for r in range(3):
        nd = lax.rsqrt(jnp.maximum(dgm_ref[2 * r + 1], 1.0))
        h2 = h2 + (acc_ref[0, r] + acc_ref[1, r]) * nd + b2_ref[r][None, :]
    o_ref[...] = h2


_cmb2 = pl.pallas_call(
    _cmb2_body,
    grid=(NPQ // RBQ,),
    in_specs=[
        pl.BlockSpec((NC, 3, RBQ, 128), lambda i: (0, 0, i, 0)),
        pl.BlockSpec((6, RBQ, 128), lambda i: (0, i, 0)),
        pl.BlockSpec((3, 128), lambda i: (0, 0)),
    ],
    out_specs=pl.BlockSpec((RBQ, 128), lambda i: (i, 0)),
    out_shape=jax.ShapeDtypeStruct((NPQ, 128), _f32),
)


def kernel(x, edge_index_rel0, edge_index_rel1, edge_index_rel2,
           pred_edge_index,
           W1_0, b1_0, W1_1, b1_1, W1_2, b1_2,
           W2_0, b2_0, W2_1, b2_1, W2_2, b2_2):
    e0s, e0d = edge_index_rel0[0], edge_index_rel0[1]
    e1s, e1d = edge_index_rel1[0], edge_index_rel1[1]
    e2s, e2d = edge_index_rel2[0], edge_index_rel2[1]
    eis = (e0s, e0d, e1s, e1d, e2s, e2d)
    zvec = jnp.zeros((RT,), _f32)
    degp = _deg_kernel(*eis, zvec)
    deg6 = degp[0] + degp[1]
    degcol = deg6.T
    degm = jnp.broadcast_to(deg6[:, :, None],
                            (6, NP, 32)).reshape(6, NPQ, 128)

    xp = jnp.pad(x, ((0, NP - N), (0, 0)))
    w1s = jnp.stack([W1_0, W1_1, W1_2])
    t1 = _mm1(xp, degcol, w1s)

    z32 = jnp.zeros((RT, DOUT), _f32)
    acc1 = _scatter_l1(t1, *eis, z32)

    eye4 = jnp.eye(4, dtype=_f32)
    b1p = jnp.stack([jnp.stack([jnp.tile(b[:DOUT], 4), jnp.tile(b[DOUT:], 4)])
                     for b in (b1_0, b1_1, b1_2)])
    w2bd = jnp.stack([jnp.stack([jnp.kron(eye4, w[:DOUT]),
                                 jnp.kron(eye4, w[DOUT:])])
                      for w in (W2_0, W2_1, W2_2)])
    acc1v = acc1.reshape(NC, 6, NPQ, 128)
    t2p = _cmb1(acc1v, degm, b1p, w2bd)
    t2 = t2p.reshape(3, NP, DOUT)

    acc2 = _scatter_l2(t2, *eis, z32)

    b2p = jnp.stack([jnp.tile(b, 4) for b in (b2_0, b2_1, b2_2)])
    acc2v = acc2.reshape(NC, 3, NPQ, 128)
    h2p = _cmb2(acc2v, degm, b2p)
    h2 = h2p.reshape(NP, DOUT)

    sc = _pred_kernel(h2, pred_edge_index[0], pred_edge_index[1])
    return sc.reshape(EP, 1)


# speedup vs baseline: 1.0558x; 1.0434x over previous
import functools

import jax
import jax.numpy as jnp
from jax import lax
from jax.experimental import pallas as pl
from jax.experimental.pallas import tpu as pltpu
from jax.experimental.pallas import tpu_sc as plsc

N = 10000
E = 320000
EP = 100000
DIN, DH, DOUT = 128, 64, 32
NP = 10240
NC, NS = 2, 16
NW = NC * NS
RT = NP // NS
ECJ = 125
EJ = 8
EC = ECJ * EJ
EW = E // NW
NCH = EW // EC
ERWS = E // ECJ
PRWS = EP // ECJ
PCH = PRWS // EJ
RB = 2048
NB = NP // RB

_MESH = plsc.VectorSubcoreMesh(
    core_axis_name="c", subcore_axis_name="s", num_cores=NC, num_subcores=NS)
_SC_PARAMS = pltpu.CompilerParams(use_tc_tiling_on_sc=False)
_SC_PARAMS_NL = pltpu.CompilerParams(use_tc_tiling_on_sc=False,
                                     needs_layout_passes=False)

_f32 = jnp.float32
_i32 = jnp.int32


@functools.partial(
    pl.kernel,
    out_type=jax.ShapeDtypeStruct((NC, 6, NP), _f32),
    mesh=_MESH,
    compiler_params=_SC_PARAMS,
    scratch_types=[
        pltpu.VMEM((1024,), _f32),
        pltpu.VMEM((RT,), _f32),
    ] + [pltpu.VMEM((EC,), _i32) for _ in range(12)]
      + [pltpu.VMEM_SHARED((NP,), _f32) for _ in range(6)]
      + [pltpu.SemaphoreType.DMA, pltpu.SemaphoreType.DMA],
)
def _deg_kernel(ei0, ei1, ei2, zvec, out, ones, stage, *rest):
    idxab = rest[:12]
    shs = rest[12:18]
    sa, sb = rest[18:]
    cid = lax.axis_index("c")
    sid = lax.axis_index("s")
    wid = cid * NS + sid
    for i in range(64):
        ones[pl.ds(16 * i, 16)] = jnp.full((16,), 1.0, _f32)
    ro0 = pl.multiple_of(sid * RT, 8)
    pltpu.sync_copy(zvec, stage)
    for sh in shs:
        pltpu.sync_copy(stage, sh.at[pl.ds(ro0, RT)])
    plsc.subcore_barrier()
    eisrc = (ei0, ei0, ei1, ei1, ei2, ei2)
    whs = (0, 1, 0, 1, 0, 1)

    def halfchunk(k, bufs, ssem, drain_first):
        eo = pl.multiple_of(wid * EW + k * EC, 8)
        if drain_first:
            for p in range(6):
                pltpu.make_async_copy(ones.at[pl.ds(0, EC)],
                                      shs[p].at[bufs[p]], ssem).wait()
        for p in range(6):
            pltpu.sync_copy(eisrc[p].at[whs[p], pl.ds(eo, EC)], bufs[p])
        for p in range(6):
            pltpu.async_copy(ones.at[pl.ds(0, EC)], shs[p].at[bufs[p]], ssem,
                             add=True)

    def chunk2(k2, carry):
        halfchunk(2 * k2, idxab[:6], sa, True)
        halfchunk(2 * k2 + 1, idxab[6:], sb, True)
        return carry

    halfchunk(0, idxab[:6], sa, False)
    halfchunk(1, idxab[6:], sb, False)

    def chunk2w(k2, carry):
        return chunk2(k2 + 1, carry)

    lax.fori_loop(0, NCH // 2 - 1, chunk2w, 0)
    for bufs, ssem in ((idxab[:6], sa), (idxab[6:], sb)):
        for p in range(6):
            pltpu.make_async_copy(ones.at[pl.ds(0, EC)],
                                  shs[p].at[bufs[p]], ssem).wait()
    plsc.subcore_barrier()
    for jj, sh in enumerate(shs):
        pltpu.sync_copy(sh.at[pl.ds(ro0, RT)], stage)
        pltpu.sync_copy(stage, out.at[cid, jj, pl.ds(ro0, RT)])


def _make_scatter(D, ei_of):
    ntab = len(ei_of)

    @functools.partial(
        pl.kernel,
        out_type=jax.ShapeDtypeStruct((NC, ntab, NP, D), _f32),
        mesh=_MESH,
        compiler_params=_SC_PARAMS,
        scratch_types=[
            pltpu.VMEM((EC,), _i32),
            pltpu.VMEM((EC,), _i32),
            pltpu.VMEM((EC, D), _f32),
            pltpu.VMEM((EC,), _i32),
            pltpu.VMEM((EC,), _i32),
            pltpu.VMEM((EC, D), _f32),
            pltpu.VMEM((RT, D), _f32),
            pltpu.VMEM_SHARED((NP, D), _f32),
            pltpu.SemaphoreType.DMA,
            pltpu.SemaphoreType.DMA,
            pltpu.SemaphoreType.DMA,
            pltpu.SemaphoreType.DMA,
        ],
    )
    def k(tabs3, e0, e1, e2, zrows, acc_out, *scr):
        tabs = [tabs3.at[t] for t in range(ntab)]
        eis = (e0, e1, e2)
        (isa, ida, rwa, isb, idb, rwb, stage, ash,
         gsa, gsb, ssa, ssb) = scr
        cid = lax.axis_index("c")
        sid = lax.axis_index("s")
        wid = cid * NS + sid
        ro0 = pl.multiple_of(sid * RT, 8)

        def load_idx(er, isx, idx_, k_):
            eo = pl.multiple_of(wid * EW + k_ * EC, 8)
            pltpu.sync_copy(er.at[0, pl.ds(eo, EC)], isx)
            pltpu.sync_copy(er.at[1, pl.ds(eo, EC)], idx_)

        def fire_g(tr, isx, rw, gs):
            pltpu.async_copy(tr.at[isx], rw, gs)

        def drain_g(tr, isx, rw, gs):
            pltpu.make_async_copy(tr.at[isx], rw, gs).wait()

        def fire_s(idx_, rw, ss):
            pltpu.async_copy(rw, ash.at[idx_], ss, add=True)

        def drain_s(idx_, rw, ss):
            pltpu.make_async_copy(rw, ash.at[idx_], ss).wait()

        for t in range(ntab):
            tr = tabs[t]
            er = eis[ei_of[t]]
            load_idx(er, isa, ida, 0)
            fire_g(tr, isa, rwa, gsa)
            pltpu.sync_copy(zrows, stage)
            pltpu.sync_copy(stage, ash.at[pl.ds(ro0, RT), :])
            plsc.subcore_barrier()

            def body(k2, carry, tr=tr, er=er):
                k_ = 2 * k2
                load_idx(er, isb, idb, k_ + 1)
                drain_g(tr, isa, rwa, gsa)
                fire_s(ida, rwa, ssa)
                fire_g(tr, isb, rwb, gsb)
                drain_s(ida, rwa, ssa)

                @pl.when(k_ + 2 < NCH)
                def _():
                    load_idx(er, isa, ida, k_ + 2)
                    fire_g(tr, isa, rwa, gsa)

                drain_g(tr, isb, rwb, gsb)
                fire_s(idb, rwb, ssb)
                drain_s(idb, rwb, ssb)
                return carry

            lax.fori_loop(0, NCH // 2, body, 0)
            plsc.subcore_barrier()
            pltpu.sync_copy(ash.at[pl.ds(ro0, RT), :], stage)
            pltpu.sync_copy(stage, acc_out.at[cid, t, pl.ds(ro0, RT), :])

    return k


_scatter_l1 = _make_scatter(DOUT, (0, 0, 1, 1, 2, 2))
_scatter_l2 = _make_scatter(DOUT, (0, 1, 2))


PH0 = 512
PH1 = EC - PH0


@functools.partial(
    pl.kernel,
    out_type=jax.ShapeDtypeStruct((EP,), _f32),
    mesh=_MESH,
    compiler_params=_SC_PARAMS_NL,
    scratch_types=[
        pltpu.VMEM((EC,), _i32),
        pltpu.VMEM((EC,), _i32),
        pltpu.VMEM((EC, DOUT), _f32),
        pltpu.VMEM((EC, DOUT), _f32),
        pltpu.VMEM((1024,), _f32),
        pltpu.SemaphoreType.DMA,
        pltpu.SemaphoreType.DMA,
    ],
)
def _pred_kernel(h2, pei, out, idxu, idxv, ur, vr, sbuf, sm0, sm1):
    cid = lax.axis_index("c")
    sid = lax.axis_index("s")
    wid = cid * NS + sid
    lane15 = lax.iota(_i32, 16) == 15
    halves = ((0, PH0, sm0), (PH0, PH1, sm1))

    def gpair(st, ln, sm):
        pltpu.async_copy(h2.at[idxu.at[pl.ds(st, ln)]],
                         ur.at[pl.ds(st, ln), :], sm)
        pltpu.async_copy(h2.at[idxv.at[pl.ds(st, ln)]],
                         vr.at[pl.ds(st, ln), :], sm)

    def dpair(st, ln, sm):
        pltpu.make_async_copy(h2.at[idxu.at[pl.ds(st, ln)]],
                              ur.at[pl.ds(st, ln), :], sm).wait()
        pltpu.make_async_copy(h2.at[idxv.at[pl.ds(st, ln)]],
                              vr.at[pl.ds(st, ln), :], sm).wait()

    def do_chunk(kk):
        eo = pl.multiple_of((wid + NW * kk) * EC, 8)
        pltpu.sync_copy(pei.at[0, pl.ds(eo, EC)], idxu)
        pltpu.sync_copy(pei.at[1, pl.ds(eo, EC)], idxv)
        for st, ln, sm in halves:
            gpair(st, ln, sm)

        def edge4(i, carry):
            for q in range(4):
                e = 4 * i + q
                p = (ur[e, pl.ds(0, 16)] * vr[e, pl.ds(0, 16)]
                     + ur[e, pl.ds(16, 16)] * vr[e, pl.ds(16, 16)])
                cs = plsc.cumsum(p)
                plsc.store_scatter(sbuf, [jnp.full((16,), e, _i32)], cs,
                                   mask=lane15)
            return carry

        dpair(0, PH0, sm0)
        lax.fori_loop(0, PH0 // 4, edge4, 0)
        dpair(PH0, PH1, sm1)
        lax.fori_loop(PH0 // 4, EC // 4, edge4, 0)
        pltpu.sync_copy(sbuf.at[pl.ds(0, EC)], out.at[pl.ds(eo, EC)])

    for kk in range(4):
        if (kk + 1) * NW <= PCH:
            do_chunk(kk)
        else:
            @pl.when(wid + NW * kk < PCH)
            def _():
                do_chunk(kk)


def _mm1_body(x_ref, dc_ref, w_ref, o_ref):
    for r in range(3):
        ns = lax.rsqrt(jnp.maximum(dc_ref[:, 2 * r:2 * r + 1], 1.0))
        xs = x_ref[...] * ns
        for h in range(2):
            o_ref[2 * r + h] = jnp.dot(
                xs, w_ref[r, :, h * DOUT:(h + 1) * DOUT],
                preferred_element_type=_f32)


_mm1 = pl.pallas_call(
    _mm1_body,
    grid=(NB,),
    in_specs=[
        pl.BlockSpec((RB, DIN), lambda i: (i, 0)),
        pl.BlockSpec((RB, 6), lambda i: (i, 0)),
        pl.BlockSpec((3, DIN, DH), lambda i: (0, 0, 0)),
    ],
    out_specs=pl.BlockSpec((6, RB, DOUT), lambda i: (0, i, 0)),
    out_shape=jax.ShapeDtypeStruct((6, NP, DOUT), _f32),
)


NPQ = NP // 4
RBQ = 256


def _cmb1_body(acc_ref, dgm_ref, b1_ref, w2_ref, o_ref):
    hp = []
    for half in range(2):
        h = jnp.zeros((RBQ, 128), _f32)
        for r in range(3):
            nd = lax.rsqrt(jnp.maximum(dgm_ref[2 * r + 1], 1.0))
            h = h + (acc_ref[0, 2 * r + half] + acc_ref[1, 2 * r + half]) \
                * nd + b1_ref[r, half][None, :]
        hp.append(jnp.maximum(h, 0.0))
    for ro in range(3):
        ns = lax.rsqrt(jnp.maximum(dgm_ref[2 * ro], 1.0))
        o_ref[ro] = (
            jnp.dot(hp[0] * ns, w2_ref[ro, 0], preferred_element_type=_f32)
            + jnp.dot(hp[1] * ns, w2_ref[ro, 1], preferred_element_type=_f32))


_cmb1 = pl.pallas_call(
    _cmb1_body,
    grid=(NPQ // RBQ,),
    in_specs=[
        pl.BlockSpec((NC, 6, RBQ, 128), lambda i: (0, 0, i, 0)),
        pl.BlockSpec((6, RBQ, 128), lambda i: (0, i, 0)),
        pl.BlockSpec((3, 2, 128), lambda i: (0, 0, 0)),
        pl.BlockSpec((3, 2, 128, 128), lambda i: (0, 0, 0, 0)),
    ],
    out_specs=pl.BlockSpec((3, RBQ, 128), lambda i: (0, i, 0)),
    out_shape=jax.ShapeDtypeStruct((3, NPQ, 128), _f32),
)


def _cmb2_body(acc_ref, dgm_ref, b2_ref, o_ref):
    h2 = jnp.zeros((RBQ, 128), _f32)
    for r in range(3):
        nd = lax.rsqrt(jnp.maximum(dgm_ref[2 * r + 1], 1.0))
        h2 = h2 + (acc_ref[0, r] + acc_ref[1, r]) * nd + b2_ref[r][None, :]
    o_ref[...] = h2


_cmb2 = pl.pallas_call(
    _cmb2_body,
    grid=(NPQ // RBQ,),
    in_specs=[
        pl.BlockSpec((NC, 3, RBQ, 128), lambda i: (0, 0, i, 0)),
        pl.BlockSpec((6, RBQ, 128), lambda i: (0, i, 0)),
        pl.BlockSpec((3, 128), lambda i: (0, 0)),
    ],
    out_specs=pl.BlockSpec((RBQ, 128), lambda i: (i, 0)),
    out_shape=jax.ShapeDtypeStruct((NPQ, 128), _f32),
)


def kernel(x, edge_index_rel0, edge_index_rel1, edge_index_rel2,
           pred_edge_index,
           W1_0, b1_0, W1_1, b1_1, W1_2, b1_2,
           W2_0, b2_0, W2_1, b2_1, W2_2, b2_2):
    ei0 = edge_index_rel0
    ei1 = edge_index_rel1
    ei2 = edge_index_rel2
    zvec = jnp.zeros((RT,), _f32)
    degp = _deg_kernel(ei0, ei1, ei2, zvec)
    deg6 = degp[0] + degp[1]
    degcol = deg6.T
    degm = jnp.broadcast_to(deg6[:, :, None],
                            (6, NP, 32)).reshape(6, NPQ, 128)

    xp = jnp.pad(x, ((0, NP - N), (0, 0)))
    w1s = jnp.stack([W1_0, W1_1, W1_2])
    t1 = _mm1(xp, degcol, w1s)

    z32 = jnp.zeros((RT, DOUT), _f32)
    acc1 = _scatter_l1(t1, ei0, ei1, ei2, z32)

    eye4 = jnp.eye(4, dtype=_f32)
    b1p = jnp.stack([jnp.stack([jnp.tile(b[:DOUT], 4), jnp.tile(b[DOUT:], 4)])
                     for b in (b1_0, b1_1, b1_2)])
    w2bd = jnp.stack([jnp.stack([jnp.kron(eye4, w[:DOUT]),
                                 jnp.kron(eye4, w[DOUT:])])
                      for w in (W2_0, W2_1, W2_2)])
    acc1v = acc1.reshape(NC, 6, NPQ, 128)
    t2p = _cmb1(acc1v, degm, b1p, w2bd)
    t2 = t2p.reshape(3, NP, DOUT)

    acc2 = _scatter_l2(t2, ei0, ei1, ei2, z32)

    b2p = jnp.stack([jnp.tile(b, 4) for b in (b2_0, b2_1, b2_2)])
    acc2v = acc2.reshape(NC, 3, NPQ, 128)
    h2p = _cmb2(acc2v, degm, b2p)
    h2 = h2p.reshape(NP, DOUT)

    sc = _pred_kernel(h2, pred_edge_index)
    return sc.reshape(EP, 1)
